# baseline (speedup 1.0000x reference)
import functools

import jax
import jax.numpy as jnp
from jax import lax
from jax.experimental import pallas as pl
from jax.experimental.pallas import tpu as pltpu
from jax.experimental.pallas import tpu_sc as plsc

N_TRAINABLE = 100000
EMBED = 32
LANES = 16
CHUNK = 1024


@functools.cache
def _make_kernel(total_rows):
    info = plsc.get_sparse_core_info()
    nw = info.num_cores * info.num_subcores
    rows_per_w = total_rows // nw
    n_chunks = rows_per_w // CHUNK
    trash = total_rows

    mesh = plsc.VectorSubcoreMesh(core_axis_name="c", subcore_axis_name="s")

    @functools.partial(
        pl.kernel,
        out_type=jax.ShapeDtypeStruct((total_rows + 8, EMBED), jnp.float32),
        mesh=mesh,
        scratch_types=[
            pltpu.VMEM((CHUNK,), jnp.int32),
            pltpu.VMEM((CHUNK,), jnp.int32),
            pltpu.VMEM((CHUNK,), jnp.int32),
            pltpu.VMEM((CHUNK,), jnp.int32),
            pltpu.VMEM((CHUNK, EMBED), jnp.float32),
            pltpu.VMEM((CHUNK, EMBED), jnp.float32),
            pltpu.SemaphoreType.DMA,
        ],
        compiler_params=pltpu.CompilerParams(use_tc_tiling_on_sc=False),
    )
    def k(idx_hbm, wt_hbm, wf_hbm, out_hbm, idx_v, idxf_v, idxt_v, post_v,
          buff_v, buft_v, sem):
        wid = lax.axis_index("s") * info.num_cores + lax.axis_index("c")
        w_base = wid * rows_per_w
        lane = lax.iota(jnp.int32, LANES)

        def body(i, _):
            base = w_base + i * CHUNK
            pltpu.sync_copy(idx_hbm.at[pl.ds(base, CHUNK)], idx_v)
            for j in range(CHUNK // LANES):
                v = idx_v[pl.ds(j * LANES, LANES)]
                m = v < N_TRAINABLE
                idxf_v[pl.ds(j * LANES, LANES)] = jnp.where(m, 0, v - N_TRAINABLE)
                idxt_v[pl.ds(j * LANES, LANES)] = jnp.where(m, v, 0)
                post_v[pl.ds(j * LANES, LANES)] = jnp.where(
                    m, base + j * LANES + lane, trash)
            pltpu.async_copy(wf_hbm.at[idxf_v], buff_v, sem).wait()
            pltpu.sync_copy(buff_v, out_hbm.at[pl.ds(base, CHUNK)])
            pltpu.async_copy(wt_hbm.at[idxt_v], buft_v, sem).wait()
            pltpu.sync_copy(buft_v, out_hbm.at[post_v])
            return ()

        lax.fori_loop(0, n_chunks, body, ())

    return k


def kernel(idx, W_train, W_frozen):
    b, h = idx.shape
    total = b * h
    flat_idx = idx.reshape(total).astype(jnp.int32)
    out = _make_kernel(total)(flat_idx, W_train, W_frozen)
    return out[:total].reshape(b, h, EMBED)


# speedup vs baseline: 6.7372x; 6.7372x over previous
import functools

import jax
import jax.numpy as jnp
from jax import lax
from jax.experimental import pallas as pl
from jax.experimental.pallas import tpu as pltpu
from jax.experimental.pallas import tpu_sc as plsc

N_TRAINABLE = 100000
EMBED = 32
LANES = 16
CHUNK = 1024
G = 128
LISTCAP = 9 * G


@functools.cache
def _make_kernel(total_rows):
    info = plsc.get_sparse_core_info()
    nw = info.num_cores * info.num_subcores
    rows_per_w = total_rows // nw
    n_chunks = rows_per_w // CHUNK
    assert n_chunks % 2 == 0
    trash = total_rows

    mesh = plsc.VectorSubcoreMesh(core_axis_name="c", subcore_axis_name="s")

    list_t = pltpu.VMEM((LISTCAP,), jnp.int32)
    rows_t = pltpu.VMEM((LISTCAP, EMBED), jnp.float32)

    @functools.partial(
        pl.kernel,
        out_type=jax.ShapeDtypeStruct((total_rows + 8, EMBED), jnp.float32),
        mesh=mesh,
        scratch_types=[
            pltpu.VMEM((CHUNK,), jnp.int32), pltpu.VMEM((CHUNK,), jnp.int32),
            list_t, list_t, list_t, list_t,
            list_t, list_t, list_t, list_t,
            rows_t, rows_t,
            pltpu.SemaphoreType.DMA,
            pltpu.SemaphoreType.DMA,
            pltpu.SemaphoreType.DMA,
        ],
        compiler_params=pltpu.CompilerParams(
            use_tc_tiling_on_sc=False, needs_layout_passes=False),
    )
    def k(idx_hbm, wt_hbm, wf_hbm, out_hbm,
          idx0, idx1, it0, pt0, if0, pf0, it1, pt1, if1, pf1,
          rb0, rb1, isem, gsem, ssem):
        wid = lax.axis_index("s") * info.num_cores + lax.axis_index("c")
        w_base = wid * rows_per_w
        lane = lax.iota(jnp.int32, LANES)
        sets = ((idx0, it0, pt0, if0, pf0, rb0),
                (idx1, it1, pt1, if1, pf1, rb1))

        def psum(x):
            dnums = lax.GatherDimensionNumbers(
                offset_dims=(), collapsed_slice_dims=(0,),
                start_index_map=(0,))
            for sh in (1, 2, 4, 8):
                src = jnp.maximum(lane - sh, 0).reshape(LANES, 1)
                g = lax.gather(x, src, dnums, (1,),
                               mode=lax.GatherScatterMode.PROMISE_IN_BOUNDS)
                x = x + jnp.where(lane >= sh, g, 0)
            return x

        def n_blocks(cnt8):
            return (cnt8 + G - 1) // G

        def blk_off(e, cnt8):
            off = jnp.minimum(e * G, jnp.maximum(cnt8 - G, 0))
            return pl.multiple_of(off, 8)

        def rb_off(e):
            return pl.multiple_of(e * G, 8)

        def drain(sem, n, proto_src, proto_dst):
            def b(_, c):
                pltpu.make_async_copy(proto_src, proto_dst, sem).wait()
                return c
            lax.fori_loop(0, n, b, 0)

        def fire_gathers(s, cnt8t, cnt8f):
            _, it, pt, if_, pf, rb = sets[s]
            nbt = n_blocks(cnt8t)

            def gt(e, c):
                off = blk_off(e, cnt8t)
                pltpu.make_async_copy(
                    wt_hbm.at[it.at[pl.ds(off, G)]],
                    rb.at[pl.ds(rb_off(e), G)], gsem).start()
                return c
            lax.fori_loop(0, nbt, gt, 0)

            def gf(e, c):
                off = blk_off(e, cnt8f)
                pltpu.make_async_copy(
                    wf_hbm.at[if_.at[pl.ds(off, G)]],
                    rb.at[pl.ds(rb_off(nbt + e), G)], gsem).start()
                return c
            lax.fori_loop(0, n_blocks(cnt8f), gf, 0)

        def fire_scatters(s, cnt8t, cnt8f):
            _, it, pt, if_, pf, rb = sets[s]
            nbt = n_blocks(cnt8t)

            def st(e, c):
                off = blk_off(e, cnt8t)
                pltpu.make_async_copy(
                    rb.at[pl.ds(rb_off(e), G)],
                    out_hbm.at[pt.at[pl.ds(off, G)]], ssem).start()
                return c
            lax.fori_loop(0, nbt, st, 0)

            def sf(e, c):
                off = blk_off(e, cnt8f)
                pltpu.make_async_copy(
                    rb.at[pl.ds(rb_off(nbt + e), G)],
                    out_hbm.at[pf.at[pl.ds(off, G)]], ssem).start()
                return c
            lax.fori_loop(0, n_blocks(cnt8f), sf, 0)

        def compact(s, c):
            idx_v, it, pt, if_, pf, _ = sets[s]
            base = w_base + c * CHUNK
            zeros = jnp.zeros((LANES,), jnp.int32)
            for kk in range(G // LANES):
                pt[pl.ds(kk * LANES, LANES)] = jnp.full(
                    (LANES,), trash, jnp.int32)
                pf[pl.ds(kk * LANES, LANES)] = jnp.full(
                    (LANES,), trash, jnp.int32)
                it[pl.ds(kk * LANES, LANES)] = zeros
                if_[pl.ds(kk * LANES, LANES)] = zeros
            cnt_t = jnp.zeros((LANES,), jnp.int32)
            cnt_f = jnp.zeros((LANES,), jnp.int32)
            for j in range(CHUNK // LANES):
                v = idx_v[pl.ds(j * LANES, LANES)]
                m = v < N_TRAINABLE
                pos = base + j * LANES + lane
                pct = psum(jnp.where(m, 1, 0))
                dest_t = cnt_t + pct - 1
                plsc.store_scatter(it, [dest_t], v, mask=m)
                plsc.store_scatter(pt, [dest_t], pos, mask=m)
                dest_f = cnt_f + lane - pct
                plsc.store_scatter(if_, [dest_f], v - N_TRAINABLE, mask=~m)
                plsc.store_scatter(pf, [dest_f], pos, mask=~m)
                np_t = plsc.all_reduce_population_count(m)
                cnt_t = cnt_t + np_t
                cnt_f = cnt_f + LANES - np_t
            ct = cnt_t[0]
            cf = cnt_f[0]
            for cnt, il, pl_ in ((ct, it, pt), (cf, if_, pf)):
                pad = ((cnt + 7) & ~7) - cnt
                pm = lane < pad
                plsc.store_scatter(il, [cnt + lane],
                                   jnp.zeros((LANES,), jnp.int32), mask=pm)
                plsc.store_scatter(pl_, [cnt + lane],
                                   jnp.full((LANES,), trash, jnp.int32),
                                   mask=pm)
            return (ct + 7) & ~7, (cf + 7) & ~7

        def fire_idx_load(c, s):
            @pl.when(c < n_chunks)
            def _():
                pltpu.make_async_copy(
                    idx_hbm.at[pl.ds(w_base + c * CHUNK, CHUNK)],
                    sets[s][0], isem).start()

        idx_proto = (idx_hbm.at[pl.ds(w_base, CHUNK)], idx0)
        g_proto = (wt_hbm.at[pl.ds(0, G)], rb0.at[pl.ds(0, G)])
        s_proto = (rb0.at[pl.ds(0, G)], out_hbm.at[pl.ds(0, G)])

        fire_idx_load(0, 0)

        def half(p, s, carry):
            c = 2 * p + s
            t1, f1, t2, f2 = carry
            drain(ssem, n_blocks(t2) + n_blocks(f2), *s_proto)
            drain(isem, 1, *idx_proto)
            ct, cf = compact(s, c)
            fire_idx_load(c + 1, 1 - s)
            fire_gathers(s, ct, cf)
            drain(gsem, n_blocks(t1) + n_blocks(f1), *g_proto)
            fire_scatters(1 - s, t1, f1)
            return ct, cf, t1, f1

        def pair(p, carry):
            carry = half(p, 0, carry)
            carry = half(p, 1, carry)
            return carry

        z = jnp.int32(0)
        t1, f1, t2, f2 = lax.fori_loop(0, n_chunks // 2, pair,
                                       (z, z, z, z))
        drain(ssem, n_blocks(t2) + n_blocks(f2), *s_proto)
        drain(gsem, n_blocks(t1) + n_blocks(f1), *g_proto)
        fire_scatters(1, t1, f1)
        drain(ssem, n_blocks(t1) + n_blocks(f1), *s_proto)

    return k


def kernel(idx, W_train, W_frozen):
    b, h = idx.shape
    total = b * h
    flat_idx = idx.reshape(total).astype(jnp.int32)
    out = _make_kernel(total)(flat_idx, W_train, W_frozen)
    return out[:total].reshape(b, h, EMBED)


# speedup vs baseline: 6.7392x; 1.0003x over previous
import functools

import jax
import jax.numpy as jnp
from jax import lax
from jax.experimental import pallas as pl
from jax.experimental.pallas import tpu as pltpu
from jax.experimental.pallas import tpu_sc as plsc

N_TRAINABLE = 100000
EMBED = 32
LANES = 16
CHUNK = 1024
G = 128
GF = 256
LISTCAP = 9 * G
RBCAP = 1408


@functools.cache
def _make_kernel(total_rows):
    info = plsc.get_sparse_core_info()
    nw = info.num_cores * info.num_subcores
    rows_per_w = total_rows // nw
    n_chunks = rows_per_w // CHUNK
    assert n_chunks % 2 == 0
    trash = total_rows

    mesh = plsc.VectorSubcoreMesh(core_axis_name="c", subcore_axis_name="s")

    list_t = pltpu.VMEM((LISTCAP,), jnp.int32)
    rows_t = pltpu.VMEM((RBCAP, EMBED), jnp.float32)

    @functools.partial(
        pl.kernel,
        out_type=jax.ShapeDtypeStruct((total_rows + 8, EMBED), jnp.float32),
        mesh=mesh,
        scratch_types=[
            pltpu.VMEM((CHUNK,), jnp.int32), pltpu.VMEM((CHUNK,), jnp.int32),
            list_t, list_t, list_t, list_t,
            list_t, list_t, list_t, list_t,
            rows_t, rows_t,
            pltpu.SemaphoreType.DMA,
            pltpu.SemaphoreType.DMA,
            pltpu.SemaphoreType.DMA,
        ],
        compiler_params=pltpu.CompilerParams(
            use_tc_tiling_on_sc=False, needs_layout_passes=False),
    )
    def k(idx_hbm, wt_hbm, wf_hbm, out_hbm,
          idx0, idx1, it0, pt0, if0, pf0, it1, pt1, if1, pf1,
          rb0, rb1, isem, gsem, ssem):
        wid = lax.axis_index("s") * info.num_cores + lax.axis_index("c")
        w_base = wid * rows_per_w
        lane = lax.iota(jnp.int32, LANES)
        sets = ((idx0, it0, pt0, if0, pf0, rb0),
                (idx1, it1, pt1, if1, pf1, rb1))

        def psum(x):
            dnums = lax.GatherDimensionNumbers(
                offset_dims=(), collapsed_slice_dims=(0,),
                start_index_map=(0,))
            for sh in (1, 2, 4, 8):
                src = jnp.maximum(lane - sh, 0).reshape(LANES, 1)
                g = lax.gather(x, src, dnums, (1,),
                               mode=lax.GatherScatterMode.PROMISE_IN_BOUNDS)
                x = x + jnp.where(lane >= sh, g, 0)
            return x

        def n_blocks(cnt8, g=G):
            return (cnt8 + g - 1) // g

        def blk_off(e, cnt8, g=G):
            off = jnp.minimum(e * g, jnp.maximum(cnt8 - g, 0))
            return pl.multiple_of(off, 8)

        def rb_off(r):
            return pl.multiple_of(r, 8)

        def drain(sem, n, proto_src, proto_dst):
            def b(_, c):
                pltpu.make_async_copy(proto_src, proto_dst, sem).wait()
                return c
            lax.fori_loop(0, n, b, 0)

        def fire_gathers(s, cnt8t, cnt8f):
            _, it, pt, if_, pf, rb = sets[s]
            nbt = n_blocks(cnt8t)

            def gt(e, c):
                off = blk_off(e, cnt8t)
                pltpu.make_async_copy(
                    wt_hbm.at[it.at[pl.ds(off, G)]],
                    rb.at[pl.ds(rb_off(e * G), G)], gsem).start()
                return c
            lax.fori_loop(0, nbt, gt, 0)

            def gf(e, c):
                off = blk_off(e, cnt8f, GF)
                pltpu.make_async_copy(
                    wf_hbm.at[if_.at[pl.ds(off, GF)]],
                    rb.at[pl.ds(rb_off(nbt * G + e * GF), GF)], gsem).start()
                return c
            lax.fori_loop(0, n_blocks(cnt8f, GF), gf, 0)

        def fire_scatters(s, cnt8t, cnt8f):
            _, it, pt, if_, pf, rb = sets[s]
            nbt = n_blocks(cnt8t)

            def st(e, c):
                off = blk_off(e, cnt8t)
                pltpu.make_async_copy(
                    rb.at[pl.ds(rb_off(e * G), G)],
                    out_hbm.at[pt.at[pl.ds(off, G)]], ssem).start()
                return c
            lax.fori_loop(0, nbt, st, 0)

            def sf(e, c):
                off = blk_off(e, cnt8f, GF)
                pltpu.make_async_copy(
                    rb.at[pl.ds(rb_off(nbt * G + e * GF), GF)],
                    out_hbm.at[pf.at[pl.ds(off, GF)]], ssem).start()
                return c
            lax.fori_loop(0, n_blocks(cnt8f, GF), sf, 0)

        def compact(s, c):
            idx_v, it, pt, if_, pf, _ = sets[s]
            base = w_base + c * CHUNK
            zeros = jnp.zeros((LANES,), jnp.int32)
            trash_v = jnp.full((LANES,), trash, jnp.int32)
            for kk in range(GF // LANES):
                if kk < G // LANES:
                    pt[pl.ds(kk * LANES, LANES)] = trash_v
                    it[pl.ds(kk * LANES, LANES)] = zeros
                pf[pl.ds(kk * LANES, LANES)] = trash_v
                if_[pl.ds(kk * LANES, LANES)] = zeros
            cnt_t = jnp.zeros((LANES,), jnp.int32)
            cnt_f = jnp.zeros((LANES,), jnp.int32)
            for j in range(CHUNK // LANES):
                v = idx_v[pl.ds(j * LANES, LANES)]
                m = v < N_TRAINABLE
                pos = base + j * LANES + lane
                pct = psum(jnp.where(m, 1, 0))
                dest_t = cnt_t + pct - 1
                plsc.store_scatter(it, [dest_t], v, mask=m)
                plsc.store_scatter(pt, [dest_t], pos, mask=m)
                dest_f = cnt_f + lane - pct
                plsc.store_scatter(if_, [dest_f], v - N_TRAINABLE, mask=~m)
                plsc.store_scatter(pf, [dest_f], pos, mask=~m)
                np_t = plsc.all_reduce_population_count(m)
                cnt_t = cnt_t + np_t
                cnt_f = cnt_f + LANES - np_t
            ct = cnt_t[0]
            cf = cnt_f[0]
            for cnt, il, pl_ in ((ct, it, pt), (cf, if_, pf)):
                pad = ((cnt + 7) & ~7) - cnt
                pm = lane < pad
                plsc.store_scatter(il, [cnt + lane],
                                   jnp.zeros((LANES,), jnp.int32), mask=pm)
                plsc.store_scatter(pl_, [cnt + lane],
                                   jnp.full((LANES,), trash, jnp.int32),
                                   mask=pm)
            return (ct + 7) & ~7, (cf + 7) & ~7

        def fire_idx_load(c, s):
            @pl.when(c < n_chunks)
            def _():
                pltpu.make_async_copy(
                    idx_hbm.at[pl.ds(w_base + c * CHUNK, CHUNK)],
                    sets[s][0], isem).start()

        idx_proto = (idx_hbm.at[pl.ds(w_base, CHUNK)], idx0)
        g_proto = (wt_hbm.at[pl.ds(0, G)], rb0.at[pl.ds(0, G)])
        s_proto = (rb0.at[pl.ds(0, G)], out_hbm.at[pl.ds(0, G)])
        gf_proto = (wf_hbm.at[pl.ds(0, GF)], rb0.at[pl.ds(0, GF)])
        sf_proto = (rb0.at[pl.ds(0, GF)], out_hbm.at[pl.ds(0, GF)])

        fire_idx_load(0, 0)

        def half(p, s, carry):
            c = 2 * p + s
            t1, f1, t2, f2 = carry
            drain(ssem, n_blocks(t2), *s_proto)
            drain(ssem, n_blocks(f2, GF), *sf_proto)
            drain(isem, 1, *idx_proto)
            ct, cf = compact(s, c)
            fire_idx_load(c + 1, 1 - s)
            fire_gathers(s, ct, cf)
            drain(gsem, n_blocks(t1), *g_proto)
            drain(gsem, n_blocks(f1, GF), *gf_proto)
            fire_scatters(1 - s, t1, f1)
            return ct, cf, t1, f1

        def pair(p, carry):
            carry = half(p, 0, carry)
            carry = half(p, 1, carry)
            return carry

        z = jnp.int32(0)
        t1, f1, t2, f2 = lax.fori_loop(0, n_chunks // 2, pair,
                                       (z, z, z, z))
        drain(ssem, n_blocks(t2), *s_proto)
        drain(ssem, n_blocks(f2, GF), *sf_proto)
        drain(gsem, n_blocks(t1), *g_proto)
        drain(gsem, n_blocks(f1, GF), *gf_proto)
        fire_scatters(1, t1, f1)
        drain(ssem, n_blocks(t1), *s_proto)
        drain(ssem, n_blocks(f1, GF), *sf_proto)

    return k


def kernel(idx, W_train, W_frozen):
    b, h = idx.shape
    total = b * h
    flat_idx = idx.reshape(total).astype(jnp.int32)
    out = _make_kernel(total)(flat_idx, W_train, W_frozen)
    return out[:total].reshape(b, h, EMBED)


# speedup vs baseline: 13.0392x; 1.9348x over previous
import functools

import jax
import jax.numpy as jnp
from jax import lax
from jax.experimental import pallas as pl
from jax.experimental.pallas import tpu as pltpu
from jax.experimental.pallas import tpu_sc as plsc

N_TRAINABLE = 100000
EMBED = 32
LANES = 16
CHUNK = 1024
G = 128
GF = 256
LISTCAP = 9 * G
RBCAP = 1408


@functools.cache
def _make_kernel(total_rows):
    info = plsc.get_sparse_core_info()
    nw = info.num_cores * info.num_subcores
    rows_per_w = total_rows // nw
    n_chunks = rows_per_w // CHUNK
    assert n_chunks % 2 == 0

    mesh = plsc.VectorSubcoreMesh(core_axis_name="c", subcore_axis_name="s")

    list_t = pltpu.VMEM((LISTCAP,), jnp.int32)
    rows_t = pltpu.VMEM((RBCAP, EMBED), jnp.float32)

    @functools.partial(
        pl.kernel,
        out_type=jax.ShapeDtypeStruct((total_rows, EMBED), jnp.float32),
        mesh=mesh,
        scratch_types=[
            pltpu.VMEM((CHUNK,), jnp.int32), pltpu.VMEM((CHUNK,), jnp.int32),
            list_t, list_t, list_t, list_t,
            list_t, list_t, list_t, list_t,
            rows_t, rows_t,
            pltpu.SemaphoreType.DMA,
            pltpu.SemaphoreType.DMA,
            pltpu.SemaphoreType.DMA,
        ],
        compiler_params=pltpu.CompilerParams(
            use_tc_tiling_on_sc=False, needs_layout_passes=False),
    )
    def k(idx_hbm, wt_hbm, wf_hbm, out_hbm,
          idx0, idx1, it0, pt0, if0, pf0, it1, pt1, if1, pf1,
          rb0, rb1, isem, gsem, ssem):
        wid = lax.axis_index("s") * info.num_cores + lax.axis_index("c")
        w_base = wid * rows_per_w
        lane = lax.iota(jnp.int32, LANES)
        sets = ((idx0, it0, pt0, if0, pf0, rb0),
                (idx1, it1, pt1, if1, pf1, rb1))

        def psum(x):
            dnums = lax.GatherDimensionNumbers(
                offset_dims=(), collapsed_slice_dims=(0,),
                start_index_map=(0,))
            for sh in (1, 2, 4, 8):
                src = jnp.maximum(lane - sh, 0).reshape(LANES, 1)
                g = lax.gather(x, src, dnums, (1,),
                               mode=lax.GatherScatterMode.PROMISE_IN_BOUNDS)
                x = x + jnp.where(lane >= sh, g, 0)
            return x

        def n_blocks(cnt8, g=G):
            return (cnt8 + g - 1) // g

        def blk_off(e, cnt8, g=G):
            off = jnp.minimum(e * g, jnp.maximum(cnt8 - g, 0))
            return pl.multiple_of(off, 8)

        def rb_off(r):
            return pl.multiple_of(r, 8)

        def drain(sem, n, proto_src, proto_dst):
            def b(_, c):
                pltpu.make_async_copy(proto_src, proto_dst, sem).wait()
                return c
            lax.fori_loop(0, n, b, 0)

        def fire_gathers(s, cnt8t, cnt8f):
            _, it, pt, if_, pf, rb = sets[s]
            nbt = n_blocks(cnt8t)

            def gt(e, c):
                off = blk_off(e, cnt8t)
                pltpu.make_async_copy(
                    wt_hbm.at[it.at[pl.ds(off, G)]],
                    rb.at[pl.ds(rb_off(e * G), G)], gsem).start()
                return c
            lax.fori_loop(0, nbt, gt, 0)

            def gf(e, c):
                off = blk_off(e, cnt8f, GF)
                pltpu.make_async_copy(
                    wf_hbm.at[if_.at[pl.ds(off, GF)]],
                    rb.at[pl.ds(rb_off(nbt * G + e * GF), GF)], gsem).start()
                return c
            lax.fori_loop(0, n_blocks(cnt8f, GF), gf, 0)

        def fire_scatters(s, cnt8t, cnt8f):
            _, it, pt, if_, pf, rb = sets[s]
            nbt = n_blocks(cnt8t)

            def st(e, c):
                off = blk_off(e, cnt8t)
                pltpu.make_async_copy(
                    rb.at[pl.ds(rb_off(e * G), G)],
                    out_hbm.at[pt.at[pl.ds(off, G)]], ssem).start()
                return c
            lax.fori_loop(0, nbt, st, 0)

            def sf(e, c):
                off = blk_off(e, cnt8f, GF)
                pltpu.make_async_copy(
                    rb.at[pl.ds(rb_off(nbt * G + e * GF), GF)],
                    out_hbm.at[pf.at[pl.ds(off, GF)]], ssem).start()
                return c
            lax.fori_loop(0, n_blocks(cnt8f, GF), sf, 0)

        def compact(s, c):
            idx_v, it, pt, if_, pf, _ = sets[s]
            base = w_base + c * CHUNK
            cnt_t = jnp.zeros((LANES,), jnp.int32)
            cnt_f = jnp.zeros((LANES,), jnp.int32)
            for j in range(CHUNK // LANES):
                v = idx_v[pl.ds(j * LANES, LANES)]
                m = v < N_TRAINABLE
                pos = base + j * LANES + lane
                pct = psum(jnp.where(m, 1, 0))
                dest_t = cnt_t + pct - 1
                plsc.store_scatter(it, [dest_t], v, mask=m)
                plsc.store_scatter(pt, [dest_t], pos, mask=m)
                dest_f = cnt_f + lane - pct
                plsc.store_scatter(if_, [dest_f], v - N_TRAINABLE, mask=~m)
                plsc.store_scatter(pf, [dest_f], pos, mask=~m)
                np_t = plsc.all_reduce_population_count(m)
                cnt_t = cnt_t + np_t
                cnt_f = cnt_f + LANES - np_t
            ct = cnt_t[0]
            cf = cnt_f[0]
            zero_i = jnp.zeros((LANES,), jnp.int32)
            for cnt, il, pl_, g_side in ((ct, it, pt, G), (cf, if_, pf, GF)):
                i0 = plsc.load_gather(il, [zero_i])
                p0 = plsc.load_gather(pl_, [zero_i])
                cnt8 = (cnt + 7) & ~7
                pm = lane < cnt8 - cnt
                plsc.store_scatter(il, [cnt + lane], i0, mask=pm)
                plsc.store_scatter(pl_, [cnt + lane], p0, mask=pm)
                for kk in range(g_side // LANES):
                    off = kk * LANES + lane
                    um = off >= cnt8
                    plsc.store_scatter(il, [off], i0, mask=um)
                    plsc.store_scatter(pl_, [off], p0, mask=um)
            return (ct + 7) & ~7, (cf + 7) & ~7

        def fire_idx_load(c, s):
            @pl.when(c < n_chunks)
            def _():
                pltpu.make_async_copy(
                    idx_hbm.at[pl.ds(w_base + c * CHUNK, CHUNK)],
                    sets[s][0], isem).start()

        idx_proto = (idx_hbm.at[pl.ds(w_base, CHUNK)], idx0)
        g_proto = (wt_hbm.at[pl.ds(0, G)], rb0.at[pl.ds(0, G)])
        s_proto = (rb0.at[pl.ds(0, G)], out_hbm.at[pl.ds(0, G)])
        gf_proto = (wf_hbm.at[pl.ds(0, GF)], rb0.at[pl.ds(0, GF)])
        sf_proto = (rb0.at[pl.ds(0, GF)], out_hbm.at[pl.ds(0, GF)])

        fire_idx_load(0, 0)

        def half(p, s, carry):
            c = 2 * p + s
            t1, f1, t2, f2 = carry
            drain(ssem, n_blocks(t2), *s_proto)
            drain(ssem, n_blocks(f2, GF), *sf_proto)
            drain(isem, 1, *idx_proto)
            ct, cf = compact(s, c)
            fire_idx_load(c + 1, 1 - s)
            fire_gathers(s, ct, cf)
            drain(gsem, n_blocks(t1), *g_proto)
            drain(gsem, n_blocks(f1, GF), *gf_proto)
            fire_scatters(1 - s, t1, f1)
            return ct, cf, t1, f1

        def pair(p, carry):
            carry = half(p, 0, carry)
            carry = half(p, 1, carry)
            return carry

        z = jnp.int32(0)
        t1, f1, t2, f2 = lax.fori_loop(0, n_chunks // 2, pair,
                                       (z, z, z, z))
        drain(ssem, n_blocks(t2), *s_proto)
        drain(ssem, n_blocks(f2, GF), *sf_proto)
        drain(gsem, n_blocks(t1), *g_proto)
        drain(gsem, n_blocks(f1, GF), *gf_proto)
        fire_scatters(1, t1, f1)
        drain(ssem, n_blocks(t1), *s_proto)
        drain(ssem, n_blocks(f1, GF), *sf_proto)

    return k


def kernel(idx, W_train, W_frozen):
    b, h = idx.shape
    total = b * h
    flat_idx = idx.reshape(total).astype(jnp.int32)
    out = _make_kernel(total)(flat_idx, W_train, W_frozen)
    return out.reshape(b, h, EMBED)
